# Initial kernel scaffold; baseline (speedup 1.0000x reference)
import jax
import jax.numpy as jnp
from jax.experimental import pallas as pl


def kernel(h, edge_index, W, b, ln_gamma, ln_beta):
    raise NotImplementedError("write your pallas kernel here")


# speedup vs baseline: 1.3450x; 1.3450x over previous
import functools

import jax
import jax.numpy as jnp
from jax import lax
from jax.experimental import pallas as pl
from jax.experimental.pallas import tpu as pltpu
from jax.experimental.pallas import tpu_sc as plsc

N = 10000
E = 320000
D = 128
NC = 2
NS = 16
NW = NC * NS
FS = D // NW
CH = 8000
NCH = E // CH


def _lane_take(x, perm):
  dnums = lax.GatherDimensionNumbers(
      offset_dims=(), collapsed_slice_dims=(0,), start_index_map=(0,))
  return lax.gather(
      x, perm[:, None], dimension_numbers=dnums, slice_sizes=(1,),
      mode=lax.GatherScatterMode.PROMISE_IN_BOUNDS)


def _sc_segment_max(ht, src, dst, agg_init):
  mesh = plsc.VectorSubcoreMesh(
      core_axis_name="c", subcore_axis_name="s", num_cores=NC,
      num_subcores=NS)

  @functools.partial(
      pl.kernel,
      out_type=jax.ShapeDtypeStruct((NW, N * FS), jnp.float32),
      mesh=mesh,
      compiler_params=pltpu.CompilerParams(
          needs_layout_passes=False, use_tc_tiling_on_sc=False),
      scratch_types=[
          pltpu.VMEM((N * FS,), jnp.float32),
          pltpu.VMEM((N * FS,), jnp.float32),
          pltpu.VMEM((CH,), jnp.int32),
          pltpu.VMEM((CH,), jnp.int32),
      ],
  )
  def k(ht_hbm, src_hbm, dst_hbm, init_hbm, out_hbm, h_v, agg_v, src_v,
        dst_v):
    wid = lax.axis_index("s") * NC + lax.axis_index("c")
    pltpu.sync_copy(ht_hbm.at[wid], h_v)
    pltpu.sync_copy(init_hbm, agg_v)

    iota = lax.iota(jnp.int32, 16)
    lane4 = iota & 3
    rep = iota >> 2

    def grp(g, _):
      ridx = g * 4 + rep
      s = plsc.load_gather(src_v, [ridx])
      d = plsc.load_gather(dst_v, [ridx])
      m = plsc.load_gather(h_v, [(s << 2) | lane4])
      val = m
      for r in (4, 8, 12):
        perm = (iota + r) & 15
        d2 = _lane_take(d, perm)
        v2 = _lane_take(m, perm)
        val = jnp.where(d2 == d, jnp.maximum(val, v2), val)
      aidx = (d << 2) | lane4
      cur = plsc.load_gather(agg_v, [aidx])
      plsc.store_scatter(agg_v, [aidx], jnp.maximum(cur, val))
      return 0

    def chunk_body(c, _):
      pltpu.sync_copy(src_hbm.at[pl.ds(c * CH, CH)], src_v)
      pltpu.sync_copy(dst_hbm.at[pl.ds(c * CH, CH)], dst_v)
      lax.fori_loop(0, CH // 4, grp, 0)
      return 0

    lax.fori_loop(0, NCH, chunk_body, 0)
    pltpu.sync_copy(agg_v, out_hbm.at[wid])

  return k(ht, src, dst, agg_init)


def _tc_post(h, agg, Wt, b2, g2, bt2):
  BLK = 1000

  def body(h_ref, a_ref, w_ref, b_ref, g_ref, bt_ref, o_ref):
    a = a_ref[...]
    a = jnp.where(a == -jnp.inf, 0.0, a)
    x = jnp.dot(h_ref[...] + a, w_ref[...],
                preferred_element_type=jnp.float32) + b_ref[...]
    x = jnp.maximum(x, 0.0)
    mu = jnp.mean(x, axis=-1, keepdims=True)
    xc = x - mu
    var = jnp.mean(xc * xc, axis=-1, keepdims=True)
    o_ref[...] = xc * lax.rsqrt(var + 1e-5) * g_ref[...] + bt_ref[...]

  return pl.pallas_call(
      body,
      grid=(N // BLK,),
      in_specs=[
          pl.BlockSpec((BLK, D), lambda i: (i, 0)),
          pl.BlockSpec((BLK, D), lambda i: (i, 0)),
          pl.BlockSpec((D, D), lambda i: (0, 0)),
          pl.BlockSpec((1, D), lambda i: (0, 0)),
          pl.BlockSpec((1, D), lambda i: (0, 0)),
          pl.BlockSpec((1, D), lambda i: (0, 0)),
      ],
      out_specs=pl.BlockSpec((BLK, D), lambda i: (i, 0)),
      out_shape=jax.ShapeDtypeStruct((N, D), jnp.float32),
  )(h, agg, Wt, b2, g2, bt2)


@jax.jit
def kernel(h, edge_index, W, b, ln_gamma, ln_beta):
  ht = h.reshape(N, NW, FS).transpose(1, 0, 2).reshape(NW, N * FS)
  src = edge_index[0]
  dst = edge_index[1]
  agg_init = jnp.full((N * FS,), -jnp.inf, jnp.float32)
  aggT = _sc_segment_max(ht, src, dst, agg_init)
  agg = aggT.reshape(NW, N, FS).transpose(1, 0, 2).reshape(N, D)
  return _tc_post(h, agg, W.T, b.reshape(1, D), ln_gamma.reshape(1, D),
                  ln_beta.reshape(1, D))
